# baseline (speedup 1.0000x reference)
import jax
import jax.numpy as jnp
from jax import lax
from jax.experimental import pallas as pl
from jax.experimental.pallas import tpu as pltpu
from jax.experimental.pallas import tpu_sc as plsc

_NC = 2
_NS = 16
_G = 5
_NB = 2
_L = 16


def _sc_body(x_hbm, w_hbm, b_hbm, o_hbm, xbuf, obuf, mbuf, wbuf, bbuf,
             insem, outsem):
    B, T, K, S = x_hbm.shape
    Z = w_hbm.shape[1]
    nchunk = T // _G
    wid = lax.axis_index("s") * _NC + lax.axis_index("c")

    pltpu.sync_copy(w_hbm, wbuf)
    pltpu.sync_copy(b_hbm, bbuf)

    def start_in(c, slot):
        pltpu.make_async_copy(
            x_hbm.at[wid, pl.ds(c * _G, _G)], xbuf.at[slot], insem.at[slot]
        ).start()

    def wait_in(slot):
        pltpu.make_async_copy(
            x_hbm.at[0, pl.ds(0, _G)], xbuf.at[slot], insem.at[slot]
        ).wait()

    def start_out(c, slot):
        pltpu.make_async_copy(
            obuf.at[slot], o_hbm.at[wid, pl.ds(c * _G, _G)], outsem.at[slot]
        ).start()

    def wait_out(slot):
        pltpu.make_async_copy(
            obuf.at[slot], o_hbm.at[0, pl.ds(0, _G)], outsem.at[slot]
        ).wait()

    for s in range(_NB):
        start_in(s, s)

    ncs = S // _L
    ncz = Z // _L
    bvec = [bbuf[pl.ds(zc * _L, _L)] for zc in range(ncz)]
    inv_k = jnp.float32(1.0 / K)

    def process(cb, slot, first):
        wait_in(slot)
        if not first:
            wait_out(slot)

        def mean_g(g, _):
            def red_k(k, accs):
                return tuple(
                    accs[c] + xbuf[slot, g, k, pl.ds(_L * c, _L)]
                    for c in range(ncs)
                )
            accs = lax.fori_loop(
                1, K, red_k,
                tuple(xbuf[slot, g, 0, pl.ds(_L * c, _L)] for c in range(ncs)),
            )
            for c in range(ncs):
                mbuf[g, pl.ds(_L * c, _L)] = accs[c] * inv_k
            return 0

        lax.fori_loop(0, _G, mean_g, 0)

        accs = [jnp.zeros((_L,), jnp.float32) for _ in range(_G * ncz)]
        for sc in range(ncs):
            mvs = [mbuf[g, pl.ds(sc * _L, _L)] for g in range(_G)]

            def mm_si(si, acc_t, sc=sc, mvs=mvs):
                idx = jnp.full((_L,), si, jnp.int32)
                s = sc * _L + si
                wv = [wbuf[s, pl.ds(zc * _L, _L)] for zc in range(ncz)]
                out = []
                dnums = lax.GatherDimensionNumbers(
                    offset_dims=(), collapsed_slice_dims=(0,),
                    start_index_map=(0,))
                for g in range(_G):
                    mg = lax.gather(
                        mvs[g], idx[:, None], dnums, (1,),
                        mode=lax.GatherScatterMode.PROMISE_IN_BOUNDS)
                    for zc in range(ncz):
                        out.append(acc_t[g * ncz + zc] + mg * wv[zc])
                return tuple(out)

            accs = list(lax.fori_loop(0, _L, mm_si, tuple(accs)))

        ys = [accs[i] + bvec[i % ncz] for i in range(_G * ncz)]

        def store_k(k, _):
            for g in range(_G):
                for zc in range(ncz):
                    obuf[slot, g, k, pl.ds(zc * _L, _L)] = ys[g * ncz + zc]
            return 0

        lax.fori_loop(0, K, store_k, 0)

        start_out(cb, slot)

        @pl.when(cb + _NB < nchunk)
        def _():
            start_in(cb + _NB, slot)

    for j in range(_NB):
        process(jnp.int32(j), j, first=True)

    def outer(cb2, _):
        for j in range(_NB):
            process(cb2 * _NB + j, j, first=False)
        return 0

    lax.fori_loop(1, nchunk // _NB, outer, 0)

    for j in range(_NB):
        wait_out(j)


def kernel(x, W, b):
    B, T, K, S = x.shape
    Z = W.shape[1]
    mesh = plsc.VectorSubcoreMesh(
        core_axis_name="c", subcore_axis_name="s",
        num_cores=_NC, num_subcores=_NS,
    )
    f = pl.kernel(
        _sc_body,
        out_type=jax.ShapeDtypeStruct((B, T, K, Z), jnp.float32),
        mesh=mesh,
        scratch_types=[
            pltpu.VMEM((_NB, _G, K, S), jnp.float32),
            pltpu.VMEM((_NB, _G, K, Z), jnp.float32),
            pltpu.VMEM((_G, S), jnp.float32),
            pltpu.VMEM((S, Z), jnp.float32),
            pltpu.VMEM((Z,), jnp.float32),
            pltpu.SemaphoreType.DMA((_NB,)),
            pltpu.SemaphoreType.DMA((_NB,)),
        ],
    )
    return f(x, W, b)


# speedup vs baseline: 1.1853x; 1.1853x over previous
import jax
import jax.numpy as jnp
from jax import lax
from jax.experimental import pallas as pl
from jax.experimental.pallas import tpu as pltpu
from jax.experimental.pallas import tpu_sc as plsc

_NC = 2
_NS = 16
_NW = _NC * _NS
_NSC = 8
_G = 5
_NB = 2
_L = 16
_TCNB = 4


def _sc_body(x_hbm, w_hbm, b_hbm, o_hbm, xbuf, obuf, mbuf, wbuf, bbuf,
             insem, outsem):
    B, T, K, S = x_hbm.shape
    Z = w_hbm.shape[1]
    wpr = _NW // _NSC
    tspan = T // wpr
    nchunk = tspan // _G
    wid = lax.axis_index("s") * _NC + lax.axis_index("c")
    row = (B - _NSC) + wid // wpr
    tbase = (wid % wpr) * tspan

    pltpu.sync_copy(w_hbm, wbuf)
    pltpu.sync_copy(b_hbm, bbuf)

    def start_in(c, slot):
        pltpu.make_async_copy(
            x_hbm.at[row, pl.ds(tbase + c * _G, _G)], xbuf.at[slot],
            insem.at[slot],
        ).start()

    def wait_in(slot):
        pltpu.make_async_copy(
            x_hbm.at[0, pl.ds(0, _G)], xbuf.at[slot], insem.at[slot]
        ).wait()

    def start_out(c, slot):
        pltpu.make_async_copy(
            obuf.at[slot],
            o_hbm.at[row - (B - _NSC), pl.ds(tbase + c * _G, _G)],
            outsem.at[slot],
        ).start()

    def wait_out(slot):
        pltpu.make_async_copy(
            obuf.at[slot], o_hbm.at[0, pl.ds(0, _G)], outsem.at[slot]
        ).wait()

    for s in range(_NB):
        start_in(s, s)

    ncs = S // _L
    ncz = Z // _L
    bvec = [bbuf[pl.ds(zc * _L, _L)] for zc in range(ncz)]
    inv_k = jnp.float32(1.0 / K)

    def process(cb, slot, first):
        wait_in(slot)
        if not first:
            wait_out(slot)

        def mean_g(g, _):
            def red_k(k, accs):
                return tuple(
                    accs[c] + xbuf[slot, g, k, pl.ds(_L * c, _L)]
                    for c in range(ncs)
                )
            accs = lax.fori_loop(
                1, K, red_k,
                tuple(xbuf[slot, g, 0, pl.ds(_L * c, _L)] for c in range(ncs)),
            )
            for c in range(ncs):
                mbuf[g, pl.ds(_L * c, _L)] = accs[c] * inv_k
            return 0

        lax.fori_loop(0, _G, mean_g, 0)

        accs = [jnp.zeros((_L,), jnp.float32) for _ in range(_G * ncz)]
        for sc in range(ncs):
            mvs = [mbuf[g, pl.ds(sc * _L, _L)] for g in range(_G)]

            def mm_si(si, acc_t, sc=sc, mvs=mvs):
                idx = jnp.full((_L,), si, jnp.int32)
                s = sc * _L + si
                wv = [wbuf[s, pl.ds(zc * _L, _L)] for zc in range(ncz)]
                out = []
                dnums = lax.GatherDimensionNumbers(
                    offset_dims=(), collapsed_slice_dims=(0,),
                    start_index_map=(0,))
                for g in range(_G):
                    mg = lax.gather(
                        mvs[g], idx[:, None], dnums, (1,),
                        mode=lax.GatherScatterMode.PROMISE_IN_BOUNDS)
                    for zc in range(ncz):
                        out.append(acc_t[g * ncz + zc] + mg * wv[zc])
                return tuple(out)

            accs = list(lax.fori_loop(0, _L, mm_si, tuple(accs)))

        ys = [accs[i] + bvec[i % ncz] for i in range(_G * ncz)]

        def store_k(k, _):
            for g in range(_G):
                for zc in range(ncz):
                    obuf[slot, g, k, pl.ds(zc * _L, _L)] = ys[g * ncz + zc]
            return 0

        lax.fori_loop(0, K, store_k, 0)

        start_out(cb, slot)

        @pl.when(cb + _NB < nchunk)
        def _():
            start_in(cb + _NB, slot)

    for j in range(_NB):
        process(jnp.int32(j), j, first=True)

    def outer(cb2, _):
        for j in range(_NB):
            process(cb2 * _NB + j, j, first=False)
        return 0

    nfull = nchunk // _NB
    lax.fori_loop(1, nfull, outer, 0)
    for j in range(nchunk - _NB * nfull):
        process(jnp.int32(_NB * nfull + j), j, first=False)

    for j in range(_NB):
        wait_out(j)


def _tc_body(x_hbm, w_ref, b_ref, o_hbm, xbuf, ybuf, insem, outsem):
    ntc = o_hbm.shape[0]

    def start_in(i, slot):
        pltpu.make_async_copy(
            x_hbm.at[i], xbuf.at[slot], insem.at[slot]
        ).start()

    def wait_in(slot):
        pltpu.make_async_copy(
            x_hbm.at[0], xbuf.at[slot], insem.at[slot]
        ).wait()

    def start_out(i, slot):
        pltpu.make_async_copy(
            ybuf.at[slot], o_hbm.at[i], outsem.at[slot]
        ).start()

    def wait_out(slot):
        pltpu.make_async_copy(
            ybuf.at[slot], o_hbm.at[0], outsem.at[slot]
        ).wait()

    for s in range(_TCNB):
        start_in(s, s)

    w = w_ref[...]
    bb = b_ref[...]

    def step(i, _):
        slot = lax.rem(i, _TCNB)
        wait_in(slot)

        @pl.when(i >= _TCNB)
        def _():
            wait_out(slot)

        m = jnp.mean(xbuf[slot], axis=1)
        ybuf[slot] = (
            jnp.dot(m, w, preferred_element_type=jnp.float32) + bb
        )
        start_out(i, slot)

        @pl.when(i + _TCNB < ntc)
        def _():
            start_in(i + _TCNB, slot)

        return 0

    lax.fori_loop(0, ntc, step, 0)

    for s in range(_TCNB):
        wait_out(s)


def kernel(x, W, b):
    B, T, K, S = x.shape
    Z = W.shape[1]
    ntc = B - _NSC

    mesh = plsc.VectorSubcoreMesh(
        core_axis_name="c", subcore_axis_name="s",
        num_cores=_NC, num_subcores=_NS,
    )
    sc_out = pl.kernel(
        _sc_body,
        out_type=jax.ShapeDtypeStruct((_NSC, T, K, Z), jnp.float32),
        mesh=mesh,
        scratch_types=[
            pltpu.VMEM((_NB, _G, K, S), jnp.float32),
            pltpu.VMEM((_NB, _G, K, Z), jnp.float32),
            pltpu.VMEM((_G, S), jnp.float32),
            pltpu.VMEM((S, Z), jnp.float32),
            pltpu.VMEM((Z,), jnp.float32),
            pltpu.SemaphoreType.DMA((_NB,)),
            pltpu.SemaphoreType.DMA((_NB,)),
        ],
    )(x, W, b)

    Wt = jnp.tile(W, (1, K))
    bt = jnp.tile(b, K).reshape(1, K * Z)
    tc_out = pl.pallas_call(
        _tc_body,
        in_specs=[
            pl.BlockSpec(memory_space=pl.ANY),
            pl.BlockSpec(memory_space=pltpu.VMEM),
            pl.BlockSpec(memory_space=pltpu.VMEM),
        ],
        out_specs=pl.BlockSpec(memory_space=pl.ANY),
        out_shape=jax.ShapeDtypeStruct((ntc, T, K * Z), jnp.float32),
        scratch_shapes=[
            pltpu.VMEM((_TCNB, T, K, S), jnp.float32),
            pltpu.VMEM((_TCNB, T, K * Z), jnp.float32),
            pltpu.SemaphoreType.DMA((_TCNB,)),
            pltpu.SemaphoreType.DMA((_TCNB,)),
        ],
    )(x, Wt, bt)

    out = jnp.concatenate([tc_out.reshape(ntc, T, K, Z), sc_out], axis=0)
    return out


# speedup vs baseline: 1.8511x; 1.5616x over previous
import jax
import jax.numpy as jnp
from jax.experimental import pallas as pl
from jax.experimental.pallas import tpu as pltpu

_NBUF = 6


def _body(x_hbm, w_ref, b_ref, o_hbm, xbuf, ybuf, insem, outsem):
    B, T, K, S = x_hbm.shape

    def start_in(i, slot):
        pltpu.make_async_copy(
            x_hbm.at[i], xbuf.at[slot], insem.at[slot]
        ).start()

    def wait_in(slot):
        pltpu.make_async_copy(
            x_hbm.at[0], xbuf.at[slot], insem.at[slot]
        ).wait()

    def start_out(i, slot):
        pltpu.make_async_copy(
            ybuf.at[slot], o_hbm.at[i], outsem.at[slot]
        ).start()

    def wait_out(slot):
        pltpu.make_async_copy(
            ybuf.at[slot], o_hbm.at[0], outsem.at[slot]
        ).wait()

    for s in range(_NBUF):
        start_in(s, s)

    w = w_ref[...]
    bb = b_ref[...]

    def step(i, _):
        slot = jax.lax.rem(i, _NBUF)
        wait_in(slot)

        @pl.when(i >= _NBUF)
        def _():
            wait_out(slot)

        m = jnp.mean(xbuf[slot], axis=1)
        ybuf[slot] = (
            jnp.dot(m, w, preferred_element_type=jnp.float32) + bb
        )
        start_out(i, slot)

        @pl.when(i + _NBUF < B)
        def _():
            start_in(i + _NBUF, slot)

        return 0

    jax.lax.fori_loop(0, B, step, 0)

    for s in range(_NBUF):
        wait_out(s)


def kernel(x, W, b):
    B, T, K, S = x.shape
    Z = W.shape[1]
    Wt = jnp.tile(W, (1, K))
    bt = jnp.tile(b, K).reshape(1, K * Z)
    out = pl.pallas_call(
        _body,
        in_specs=[
            pl.BlockSpec(memory_space=pl.ANY),
            pl.BlockSpec(memory_space=pltpu.VMEM),
            pl.BlockSpec(memory_space=pltpu.VMEM),
        ],
        out_specs=pl.BlockSpec(memory_space=pl.ANY),
        out_shape=jax.ShapeDtypeStruct((B, T, K * Z), jnp.float32),
        scratch_shapes=[
            pltpu.VMEM((_NBUF, T, K, S), jnp.float32),
            pltpu.VMEM((_NBUF, T, K * Z), jnp.float32),
            pltpu.SemaphoreType.DMA((_NBUF,)),
            pltpu.SemaphoreType.DMA((_NBUF,)),
        ],
    )(x, Wt, bt)
    return out.reshape(B, T, K, Z)


# speedup vs baseline: 1.8574x; 1.0034x over previous
import jax
import jax.numpy as jnp
from jax.experimental import pallas as pl
from jax.experimental.pallas import tpu as pltpu

_NBUF = 8


def _body(x_hbm, w_ref, b_ref, o_hbm, xbuf, ybuf, insem, outsem):
    B, T, K, S = x_hbm.shape

    def start_in(i, slot):
        pltpu.make_async_copy(
            x_hbm.at[i], xbuf.at[slot], insem.at[slot]
        ).start()

    def wait_in(slot):
        pltpu.make_async_copy(
            x_hbm.at[0], xbuf.at[slot], insem.at[slot]
        ).wait()

    def start_out(i, slot):
        pltpu.make_async_copy(
            ybuf.at[slot], o_hbm.at[i], outsem.at[slot]
        ).start()

    def wait_out(slot):
        pltpu.make_async_copy(
            ybuf.at[slot], o_hbm.at[0], outsem.at[slot]
        ).wait()

    for s in range(_NBUF):
        start_in(s, s)

    w = w_ref[...]
    bb = b_ref[...]

    def step(i, _):
        slot = jax.lax.rem(i, _NBUF)
        wait_in(slot)

        @pl.when(i >= _NBUF)
        def _():
            wait_out(slot)

        m = jnp.mean(xbuf[slot], axis=1)
        ybuf[slot] = (
            jnp.dot(m, w, preferred_element_type=jnp.float32) + bb
        )
        start_out(i, slot)

        @pl.when(i + _NBUF < B)
        def _():
            start_in(i + _NBUF, slot)

        return 0

    jax.lax.fori_loop(0, B, step, 0)

    for s in range(_NBUF):
        wait_out(s)


def kernel(x, W, b):
    B, T, K, S = x.shape
    Z = W.shape[1]
    Wt = jnp.tile(W, (1, K))
    bt = jnp.tile(b, K).reshape(1, K * Z)
    out = pl.pallas_call(
        _body,
        in_specs=[
            pl.BlockSpec(memory_space=pl.ANY),
            pl.BlockSpec(memory_space=pltpu.VMEM),
            pl.BlockSpec(memory_space=pltpu.VMEM),
        ],
        out_specs=pl.BlockSpec(memory_space=pl.ANY),
        out_shape=jax.ShapeDtypeStruct((B, T, K * Z), jnp.float32),
        scratch_shapes=[
            pltpu.VMEM((_NBUF, T, K, S), jnp.float32),
            pltpu.VMEM((_NBUF, T, K * Z), jnp.float32),
            pltpu.SemaphoreType.DMA((_NBUF,)),
            pltpu.SemaphoreType.DMA((_NBUF,)),
        ],
    )(x, Wt, bt)
    return out.reshape(B, T, K, Z)
